# baseline (speedup 1.0000x reference)
import functools

import jax
import jax.numpy as jnp
from jax import lax
from jax.experimental import pallas as pl
from jax.experimental.pallas import tpu as pltpu
from jax.experimental.pallas import tpu_sc as plsc

_VOCAB = 100000
_D = 64
_B = 4096
_T = 50
_NW = 32
_BPW = _B // _NW
_RB = 4
_L = 16
_TP = 136


def _transpose_rows(rows, tb, dvecs):
    def rbody(r, carry):
        rvec = jnp.broadcast_to(r, (_L,))
        for dc in range(_D // _L):
            v = rows[r, pl.ds(dc * _L, _L)]
            plsc.store_scatter(tb, [dvecs[dc], rvec], v)
        return carry

    lax.fori_loop(0, _BPW, rbody, 0, unroll=32)


def _transpose_idx(raw, idx_v, tvecs):
    def rbody(r, carry):
        rvec = jnp.broadcast_to(r, (_L,))
        for tc in range(_T // _L):
            v = raw[r, pl.ds(tc * _L, _L)]
            plsc.store_scatter(idx_v, [tvecs[tc], rvec], v)
        return carry

    lax.fori_loop(0, _BPW, rbody, 0, unroll=16)

    lanes = lax.iota(jnp.int32, _L)
    for t in range((_T // _L) * _L, _T):
        tv = jnp.broadcast_to(t, (_L,))
        for rc in range(_BPW // _L):
            v = plsc.load_gather(raw, [rc * _L + lanes, tv])
            idx_v[t, pl.ds(rc * _L, _L)] = v


def _emb_body(table_hbm, idx_hbm, out_hbm, idx_raw, idx_v, rows_v, tbuf,
              gsem, ssem):
    wid = lax.axis_index("s") * 2 + lax.axis_index("c")
    base = wid * _BPW
    lanes = lax.iota(jnp.int32, _L)
    dvecs = [dc * _L + lanes for dc in range(_D // _L)]
    tvecs = [tc * _L + lanes for tc in range(_T // _L + 1)]

    pltpu.sync_copy(idx_hbm.at[pl.ds(base, _BPW)], idx_raw)
    _transpose_idx(idx_raw, idx_v, tvecs)

    def fire_gather(t):
        pltpu.async_copy(
            table_hbm.at[idx_v.at[t, pl.ds(0, _BPW)]],
            rows_v.at[lax.rem(t, _RB)],
            gsem,
        )

    def wait_gather():
        pltpu.make_async_copy(
            table_hbm.at[idx_v.at[0, pl.ds(0, _BPW)]],
            rows_v.at[0],
            gsem,
        ).wait()

    def wait_store():
        pltpu.make_async_copy(
            tbuf.at[0, :, pl.ds(0, _BPW)],
            out_hbm.at[0, :, pl.ds(0, _BPW)],
            ssem,
        ).wait()

    for t in range(_RB):
        fire_gather(t)

    def body(t, carry):
        phase = lax.rem(t, 2)
        wait_gather()

        @pl.when(t >= 2)
        def _():
            wait_store()

        _transpose_rows(rows_v.at[lax.rem(t, _RB)], tbuf.at[phase], dvecs)

        pltpu.async_copy(
            tbuf.at[phase, :, pl.ds(0, _BPW)],
            out_hbm.at[t, :, pl.ds(base, _BPW)],
            ssem,
        )

        @pl.when(t + _RB < _T)
        def _():
            fire_gather(t + _RB)

        return carry

    lax.fori_loop(0, _T, body, 0)
    wait_store()
    wait_store()


_emb_call = functools.partial(
    pl.kernel,
    mesh=plsc.VectorSubcoreMesh(core_axis_name="c", subcore_axis_name="s"),
    out_type=jax.ShapeDtypeStruct((_T, _D, _B), jnp.float32),
    scratch_types=[
        pltpu.VMEM((_BPW, _T), jnp.int32),
        pltpu.VMEM((_T, _TP), jnp.int32),
        pltpu.VMEM((_RB, _BPW, 2 * _D), jnp.float32),
        pltpu.VMEM((2, _D, _BPW + 1), jnp.float32),
        pltpu.SemaphoreType.DMA,
        pltpu.SemaphoreType.DMA,
    ],
    compiler_params=pltpu.CompilerParams(
        use_tc_tiling_on_sc=False, needs_layout_passes=False
    ),
)(_emb_body)


@jax.jit
def kernel(input, emb_weight):
    table = jnp.concatenate([emb_weight, emb_weight], axis=1)
    out_t = _emb_call(table, input.astype(jnp.int32))
    return out_t.transpose(2, 0, 1)


# speedup vs baseline: 1.0710x; 1.0710x over previous
import functools

import jax
import jax.numpy as jnp
from jax import lax
from jax.experimental import pallas as pl
from jax.experimental.pallas import tpu as pltpu
from jax.experimental.pallas import tpu_sc as plsc

_VOCAB = 100000
_D = 64
_B = 4096
_T = 50
_NW = 32
_BPW = _B // _NW
_RB = 3
_L = 16
_TP = 136


def _transpose_rows(rows, tb, dvecs):
    def rbody(r, carry):
        rvec = jnp.broadcast_to(r, (_L,))
        for dc in range(_D // _L):
            v = rows[r, pl.ds(dc * _L, _L)]
            plsc.store_scatter(tb, [dvecs[dc], rvec], v)
        return carry

    lax.fori_loop(0, _BPW, rbody, 0, unroll=32)


def _transpose_idx(raw, idx_v, tvecs):
    def rbody(r, carry):
        rvec = jnp.broadcast_to(r, (_L,))
        for tc in range(_T // _L):
            v = raw[r, pl.ds(tc * _L, _L)]
            plsc.store_scatter(idx_v, [tvecs[tc], rvec], v)
        return carry

    lax.fori_loop(0, _BPW, rbody, 0, unroll=16)

    lanes = lax.iota(jnp.int32, _L)
    for t in range((_T // _L) * _L, _T):
        tv = jnp.broadcast_to(t, (_L,))
        for rc in range(_BPW // _L):
            v = plsc.load_gather(raw, [rc * _L + lanes, tv])
            idx_v[t, pl.ds(rc * _L, _L)] = v


def _emb_body(table_hbm, idx_hbm, out_hbm, idx_raw, idx_v, rows_v, tbuf,
              gsem, ssem):
    wid = lax.axis_index("s") * 2 + lax.axis_index("c")
    base = wid * _BPW
    lanes = lax.iota(jnp.int32, _L)
    dvecs = [dc * _L + lanes for dc in range(_D // _L)]
    tvecs = [tc * _L + lanes for tc in range(_T // _L + 1)]

    pltpu.sync_copy(idx_hbm.at[pl.ds(base, _BPW)], idx_raw)
    _transpose_idx(idx_raw, idx_v, tvecs)

    def fire_gather(t):
        pltpu.async_copy(
            table_hbm.at[idx_v.at[t, pl.ds(0, _BPW)]],
            rows_v.at[lax.rem(t, _RB)],
            gsem,
        )

    def wait_gather():
        pltpu.make_async_copy(
            table_hbm.at[idx_v.at[0, pl.ds(0, _BPW)]],
            rows_v.at[0],
            gsem,
        ).wait()

    def wait_store():
        pltpu.make_async_copy(
            tbuf.at[0, :, pl.ds(0, _BPW)],
            out_hbm.at[0, :, pl.ds(0, _BPW)],
            ssem,
        ).wait()

    for t in range(_RB):
        fire_gather(t)

    def body(t, carry):
        phase = lax.rem(t, 2)
        wait_gather()

        @pl.when(t >= 2)
        def _():
            wait_store()

        _transpose_rows(rows_v.at[lax.rem(t, _RB)], tbuf.at[phase], dvecs)

        pltpu.async_copy(
            tbuf.at[phase, :, pl.ds(0, _BPW)],
            out_hbm.at[t, :, pl.ds(base, _BPW)],
            ssem,
        )

        @pl.when(t + _RB < _T)
        def _():
            fire_gather(t + _RB)

        return carry

    lax.fori_loop(0, _T, body, 0)
    wait_store()
    wait_store()


_emb_call = functools.partial(
    pl.kernel,
    mesh=plsc.VectorSubcoreMesh(core_axis_name="c", subcore_axis_name="s"),
    out_type=jax.ShapeDtypeStruct((_T, _D, _B), jnp.float32),
    scratch_types=[
        pltpu.VMEM((_BPW, _T), jnp.int32),
        pltpu.VMEM((_T, _TP), jnp.int32),
        pltpu.VMEM((_RB, _BPW, 2 * _D), jnp.float32),
        pltpu.VMEM((2, _D, _BPW + 1), jnp.float32),
        pltpu.SemaphoreType.DMA,
        pltpu.SemaphoreType.DMA,
    ],
    compiler_params=pltpu.CompilerParams(
        use_tc_tiling_on_sc=False, needs_layout_passes=False
    ),
)(_emb_body)


@jax.jit
def kernel(input, emb_weight):
    table = jnp.pad(emb_weight, ((0, 0), (0, _D)))
    out_t = _emb_call(table, input.astype(jnp.int32))
    return out_t.transpose(2, 0, 1)
